# Initial kernel scaffold; baseline (speedup 1.0000x reference)
import jax
import jax.numpy as jnp
from jax.experimental import pallas as pl


def kernel(pred_logits, pred_segments, target_sizes):
    raise NotImplementedError("write your pallas kernel here")


# speedup vs baseline: 1.0651x; 1.0651x over previous
import jax
import jax.numpy as jnp
from jax.experimental import pallas as pl

POSTPROC_INS_TOPK = 100


def _sigmoid_body(logits_ref, prob_ref):
    prob_ref[...] = jax.nn.sigmoid(logits_ref[...])


def kernel(pred_logits, pred_segments, target_sizes):
    B, Q, C = pred_logits.shape
    k = min(POSTPROC_INS_TOPK, Q * C)
    prob = pl.pallas_call(
        _sigmoid_body,
        out_shape=jax.ShapeDtypeStruct((B, Q, C), jnp.float32),
        grid=(B,),
        in_specs=[pl.BlockSpec((1, Q, C), lambda b: (b, 0, 0))],
        out_specs=pl.BlockSpec((1, Q, C), lambda b: (b, 0, 0)),
    )(pred_logits)
    flat = prob.reshape(B, Q * C)
    topk_values, topk_indexes = jax.lax.top_k(flat, k)
    scores = topk_values
    topk_segments = topk_indexes // C
    labels = topk_indexes % C
    c = pred_segments[..., 0]
    w = pred_segments[..., 1]
    segs = jnp.stack([c - 0.5 * w, c + 0.5 * w], axis=-1)
    segs = jnp.take_along_axis(segs, topk_segments[:, :, None], axis=1)
    segs = segs * target_sizes[:, None, None]
    return scores, labels, segs, topk_segments


# speedup vs baseline: 3.9218x; 3.6820x over previous
import functools

import jax
import jax.numpy as jnp
from jax import lax
from jax.experimental import pallas as pl
from jax.experimental.pallas import tpu as pltpu
from jax.experimental.pallas import tpu_sc as plsc

B, Q, C = 16, 5000, 200
N = Q * C
K = 100
OUTP = 112
WLEN = N // 2
CHUNK = 20000
NCHUNK = WLEN // CHUNK
NVREG = CHUNK // 16
HB = 4096
CAP = 8192
SVCAP = 128
TA = 60
L = 16

_I16 = lambda v: jnp.full((L,), v, jnp.int32)
_F16 = lambda v: jnp.full((L,), v, jnp.float32)


def _scalar(x):
    return jnp.max(x)


def _ukey(p):
    return plsc.bitcast(_F16(1.0) - p, jnp.int32)


def _find_threshold(hist, phist, target):
    def body(i, carry):
        acc, ustar, found = carry
        hv = hist[pl.ds(i * L, L)] + phist[pl.ds(i * L, L)]
        cs = plsc.cumsum(hv)
        tot = jnp.max(cs)
        lane = jnp.sum((cs < (target - acc)).astype(jnp.int32))
        cross = jnp.logical_and(jnp.logical_not(found), acc + tot >= target)
        ustar = jnp.where(cross, i * L + lane, ustar)
        found = jnp.logical_or(found, cross)
        return acc + tot, ustar, found

    acc, ustar, found = lax.fori_loop(
        0, HB // L, body, (jnp.int32(0), jnp.int32(HB - 1), jnp.bool_(False)))
    return ustar, found


def _sc_body(probs, segf, ts, o_sc, o_lb, o_sg, o_qd,
             buf, hist, phist, cand_p, cand_i, sv_p, sv_i,
             seg_v, ts_v, cnt_v,
             ob_sc, ob_lb, ob_qd, ob_sg,
             sp_hist, sp_svp, sp_svi, sp_svc,
             sem0, sem1, sem2):
    c = lax.axis_index("c")
    s = lax.axis_index("s")
    b = c * 8 + s // 2
    half = s % 2
    wstart = half * WLEN

    iota = lax.iota(jnp.int32, L)
    ones_i = jnp.ones((L,), jnp.int32)

    pbase = b * N
    dma0 = pltpu.async_copy(probs.at[pl.ds(pbase + wstart, CHUNK)], buf.at[0], sem0)
    seg_dma = pltpu.async_copy(segf.at[pl.ds(b * 2 * Q, 2 * Q)], seg_v, sem2)
    pltpu.sync_copy(ts, ts_v)
    dma0.wait()
    dma1 = pltpu.async_copy(
        probs.at[pl.ds(pbase + wstart + CHUNK, CHUNK)], buf.at[1], sem1)

    def zero_hist(i, _):
        hist[pl.ds(i * L, L)] = jnp.zeros((L,), jnp.int32)
        return 0

    lax.fori_loop(0, HB // L, zero_hist, 0)

    def hist_body(j, _):
        p = buf[0, pl.ds(j * L, L)]
        binv = lax.shift_right_logical(_ukey(p), _I16(19))
        plsc.addupdate_scatter(hist, [binv], ones_i)
        return 0

    lax.fori_loop(0, NVREG, hist_body, 0)

    pltpu.sync_copy(hist, sp_hist.at[s])
    plsc.subcore_barrier()
    pltpu.sync_copy(sp_hist.at[s + 1 - 2 * half], phist)
    ustar, _ = _find_threshold(hist, phist, jnp.int32(TA))
    uthr = _I16((ustar + 1) << 19)

    def scan_chunk(buf_ref, cbase, off):
        def body(j, off):
            p = buf_ref[pl.ds(j * L, L)]
            m = _ukey(p) < uthr
            mi = m.astype(jnp.int32)
            pos = off + plsc.cumsum(mi) - 1
            m2 = jnp.logical_and(m, pos < _I16(CAP))
            plsc.store_scatter(cand_p, [pos], p, mask=m2)
            idxv = _I16(cbase + j * L) + iota
            plsc.store_scatter(cand_i, [pos], idxv, mask=m2)
            return off + plsc.all_reduce_population_count(m)

        return lax.fori_loop(0, NVREG, body, off)

    off = jnp.zeros((L,), jnp.int32)
    for ci in range(NCHUNK):
        if ci > 0:
            (dma1 if ci % 2 == 1 else dma0).wait()
        off = scan_chunk(buf.at[ci % 2], wstart + ci * CHUNK, off)
        if ci + 2 < NCHUNK:
            nxt = pltpu.async_copy(
                probs.at[pl.ds(pbase + wstart + (ci + 2) * CHUNK, CHUNK)],
                buf.at[ci % 2], sem0 if ci % 2 == 0 else sem1)
            if ci % 2 == 0:
                dma0 = nxt
            else:
                dma1 = nxt

    ncand = jnp.minimum(_scalar(off), jnp.int32(CAP))

    lax.fori_loop(0, HB // L, zero_hist, 0)

    def chist_body(j, _):
        p = cand_p[pl.ds(j * L, L)]
        m = (iota + j * L) < _I16(ncand)
        binv = lax.shift_right_logical(_ukey(p), _I16(19))
        plsc.addupdate_scatter(hist, [binv], ones_i, mask=m)
        return 0

    lax.fori_loop(0, (ncand + L - 1) // L, chist_body, 0)

    pltpu.sync_copy(hist, sp_hist.at[s])
    plsc.subcore_barrier()
    pltpu.sync_copy(sp_hist.at[s + 1 - 2 * half], phist)
    ustar2, _ = _find_threshold(hist, phist, jnp.int32(K))
    uthr2 = _I16((ustar2 + 1) << 19)

    def sent_fill(i, _):
        sv_p[pl.ds(i * L, L)] = _F16(-1.0)
        sv_i[pl.ds(i * L, L)] = _I16(0x7FFFFFF)
        return 0

    lax.fori_loop(0, 2 * SVCAP // L, sent_fill, 0)

    def sv_body(j, off):
        p = cand_p[pl.ds(j * L, L)]
        iv = cand_i[pl.ds(j * L, L)]
        m = jnp.logical_and(_ukey(p) < uthr2, (iota + j * L) < _I16(ncand))
        mi = m.astype(jnp.int32)
        pos = off + plsc.cumsum(mi) - 1
        m2 = jnp.logical_and(m, pos < _I16(SVCAP))
        plsc.store_scatter(sv_p, [pos], p, mask=m2)
        plsc.store_scatter(sv_i, [pos], iv, mask=m2)
        return off + plsc.all_reduce_population_count(m)

    svoff = lax.fori_loop(0, (ncand + L - 1) // L, sv_body,
                          jnp.zeros((L,), jnp.int32))
    svc = jnp.minimum(_scalar(svoff), jnp.int32(SVCAP))

    pltpu.sync_copy(sv_p.at[pl.ds(0, SVCAP)], sp_svp.at[s])
    pltpu.sync_copy(sv_i.at[pl.ds(0, SVCAP)], sp_svi.at[s])
    cnt_v[...] = jnp.full((L,), svc, jnp.int32)
    pltpu.sync_copy(cnt_v, sp_svc.at[s])
    plsc.subcore_barrier()

    @pl.when(half == 0)
    def _():
        svc8 = pl.multiple_of((svc + 7) & ~jnp.int32(7), 8)
        pltpu.sync_copy(sp_svp.at[s + 1], sv_p.at[pl.ds(svc8, SVCAP)])
        pltpu.sync_copy(sp_svi.at[s + 1], sv_i.at[pl.ds(svc8, SVCAP)])
        pltpu.sync_copy(sp_svc.at[s + 1], cnt_v)
        stot = svc8 + jnp.minimum(jnp.max(cnt_v[...]), jnp.int32(SVCAP))
        sloop = svc8 + jnp.int32(SVCAP)
        del stot

        zz16 = jnp.zeros((L,), jnp.int32)
        for i in range(OUTP // L):
            ob_sc[pl.ds(i * L, L)] = _F16(0.0)
            ob_lb[pl.ds(i * L, L)] = _I16(0)
            ob_qd[pl.ds(i * L, L)] = _I16(0)
            rows = iota + i * L
            plsc.store_scatter(ob_sg, [rows * 2], _F16(0.0))
            plsc.store_scatter(ob_sg, [rows * 2 + 1], _F16(0.0))

        seg_dma.wait()
        vl = plsc.load_gather(ts_v, [_I16(0) + b])

        def rank_vreg(i, _):
            pv = sv_p[pl.ds(i * L, L)]
            iv = sv_i[pl.ds(i * L, L)]

            def jbody(j, cnt):
                js = _I16(0) + j
                pj = plsc.load_gather(sv_p, [js])
                ij = plsc.load_gather(sv_i, [js])
                beats = jnp.logical_or(
                    pj > pv, jnp.logical_and(pj == pv, ij < iv))
                return cnt + beats.astype(jnp.int32)

            r = lax.fori_loop(0, sloop, jbody, jnp.zeros((L,), jnp.int32))
            keep = jnp.logical_and(r < _I16(K), pv > _F16(-0.5))
            lab = iv % _I16(C)
            qid = iv // _I16(C)
            plsc.store_scatter(ob_sc, [r], pv, mask=keep)
            plsc.store_scatter(ob_lb, [r], lab, mask=keep)
            plsc.store_scatter(ob_qd, [r], qid, mask=keep)
            sc_ = plsc.load_gather(seg_v, [qid * 2], mask=keep)
            sw_ = plsc.load_gather(seg_v, [qid * 2 + 1], mask=keep)
            t1 = (sc_ - 0.5 * sw_) * vl
            t2 = (sc_ + 0.5 * sw_) * vl
            plsc.store_scatter(ob_sg, [r * 2], t1, mask=keep)
            plsc.store_scatter(ob_sg, [r * 2 + 1], t2, mask=keep)
            return 0

        lax.fori_loop(0, (sloop + L - 1) // L, rank_vreg, 0)

        pltpu.sync_copy(ob_sc, o_sc.at[pl.ds(b * OUTP, OUTP)])
        pltpu.sync_copy(ob_lb, o_lb.at[pl.ds(b * OUTP, OUTP)])
        pltpu.sync_copy(ob_qd, o_qd.at[pl.ds(b * OUTP, OUTP)])
        pltpu.sync_copy(ob_sg, o_sg.at[pl.ds(b * 2 * OUTP, 2 * OUTP)])


@functools.partial(
    pl.kernel,
    out_type=[
        jax.ShapeDtypeStruct((B * OUTP,), jnp.float32),
        jax.ShapeDtypeStruct((B * OUTP,), jnp.int32),
        jax.ShapeDtypeStruct((B * OUTP * 2,), jnp.float32),
        jax.ShapeDtypeStruct((B * OUTP,), jnp.int32),
    ],
    mesh=plsc.VectorSubcoreMesh(core_axis_name="c", subcore_axis_name="s"),
    compiler_params=pltpu.CompilerParams(needs_layout_passes=False, use_tc_tiling_on_sc=False),
    scratch_types=[
        pltpu.VMEM((2, CHUNK), jnp.float32),
        pltpu.VMEM((HB,), jnp.int32),
        pltpu.VMEM((HB,), jnp.int32),
        pltpu.VMEM((CAP,), jnp.float32),
        pltpu.VMEM((CAP,), jnp.int32),
        pltpu.VMEM((2 * SVCAP,), jnp.float32),
        pltpu.VMEM((2 * SVCAP,), jnp.int32),
        pltpu.VMEM((2 * Q,), jnp.float32),
        pltpu.VMEM((L,), jnp.float32),
        pltpu.VMEM((L,), jnp.int32),
        pltpu.VMEM((OUTP,), jnp.float32),
        pltpu.VMEM((OUTP,), jnp.int32),
        pltpu.VMEM((OUTP,), jnp.int32),
        pltpu.VMEM((OUTP * 2,), jnp.float32),
        pltpu.VMEM_SHARED((16, HB), jnp.int32),
        pltpu.VMEM_SHARED((16, SVCAP), jnp.float32),
        pltpu.VMEM_SHARED((16, SVCAP), jnp.int32),
        pltpu.VMEM_SHARED((16, L), jnp.int32),
        pltpu.SemaphoreType.DMA,
        pltpu.SemaphoreType.DMA,
        pltpu.SemaphoreType.DMA,
    ],
)
def _postprocess_sc(probs, segf, ts, o_sc, o_lb, o_sg, o_qd, *scratch):
    _sc_body(probs, segf, ts, o_sc, o_lb, o_sg, o_qd, *scratch)


def kernel(pred_logits, pred_segments, target_sizes):
    probs = jax.nn.sigmoid(pred_logits).reshape(B * N)
    segf = pred_segments.reshape(B * 2 * Q)
    o_sc, o_lb, o_sg, o_qd = _postprocess_sc(probs, segf, target_sizes)
    o_sc = o_sc.reshape(B, OUTP)[:, :K]
    o_lb = o_lb.reshape(B, OUTP)[:, :K]
    o_sg = o_sg.reshape(B, OUTP, 2)[:, :K, :]
    o_qd = o_qd.reshape(B, OUTP)[:, :K]
    return (o_sc, o_lb, o_sg, o_qd)


# speedup vs baseline: 5.1407x; 1.3108x over previous
import functools

import jax
import jax.numpy as jnp
from jax import lax
from jax.experimental import pallas as pl
from jax.experimental.pallas import tpu as pltpu
from jax.experimental.pallas import tpu_sc as plsc

B, Q, C = 16, 5000, 200
K = 100
OUTP = 112
COLS = 256
SUBV = 13
PADN = Q * COLS
QPW = Q // 2
QPC = 100
CHUNKP = QPC * COLS
NCHUNK = QPW // QPC
HB = 4096
CAP = 8192
SVCAP = 128
TA = 60
L = 16

_I16 = lambda v: jnp.full((L,), v, jnp.int32)
_F16 = lambda v: jnp.full((L,), v, jnp.float32)


def _scalar(x):
    return jnp.max(x)


def _ukey(p):
    return plsc.bitcast(_F16(1.0) - p, jnp.int32)


def _find_threshold(hist, phist, target):
    def body(i, carry):
        acc, ustar, found = carry
        hv = hist[pl.ds(i * L, L)] + phist[pl.ds(i * L, L)]
        cs = plsc.cumsum(hv)
        tot = jnp.max(cs)
        lane = jnp.sum((cs < (target - acc)).astype(jnp.int32))
        cross = jnp.logical_and(jnp.logical_not(found), acc + tot >= target)
        ustar = jnp.where(cross, i * L + lane, ustar)
        found = jnp.logical_or(found, cross)
        return acc + tot, ustar, found

    acc, ustar, found = lax.fori_loop(
        0, HB // L, body, (jnp.int32(0), jnp.int32(HB - 1), jnp.bool_(False)))
    return ustar, found


def _sigmoid_pad_body(x_ref, o_ref):
    o_ref[0, :, 0:C] = jax.nn.sigmoid(x_ref[0, :, :])
    o_ref[0, :, C:COLS] = jnp.zeros((Q, COLS - C), jnp.float32)


def _sc_body(probs, segf, ts, o_sc, o_lb, o_sg, o_qd,
             buf, hist, phist, cand_p, cand_i, sv_p, sv_i,
             seg_v, ts_v, cnt_v,
             ob_sc, ob_lb, ob_qd, ob_sg,
             sp_hist, sp_svp, sp_svi, sp_svc,
             sem0, sem1, sem2):
    c = lax.axis_index("c")
    s = lax.axis_index("s")
    b = c * 8 + s // 2
    half = s % 2
    wstart = half * QPW * COLS

    iota = lax.iota(jnp.int32, L)
    ones_i = jnp.ones((L,), jnp.int32)

    pbase = b * PADN
    dma0 = pltpu.async_copy(
        probs.at[pl.ds(pbase + wstart, CHUNKP)], buf.at[0], sem0)
    seg_dma = pltpu.async_copy(segf.at[pl.ds(b * 2 * Q, 2 * Q)], seg_v, sem2)
    pltpu.sync_copy(ts, ts_v)
    dma0.wait()
    dma1 = pltpu.async_copy(
        probs.at[pl.ds(pbase + wstart + CHUNKP, CHUNKP)], buf.at[1], sem1)

    def zero_hist(i, _):
        hist[pl.ds(i * L, L)] = jnp.zeros((L,), jnp.int32)
        return 0

    lax.fori_loop(0, HB // L, zero_hist, 0)

    def hist_row(r, _):
        base = r * COLS
        for sub in range(SUBV):
            p = buf[0, pl.ds(base + sub * L, L)]
            binv = lax.shift_right_logical(_ukey(p), _I16(19))
            plsc.addupdate_scatter(hist, [binv], ones_i)
        return 0

    lax.fori_loop(0, QPC, hist_row, 0)

    pltpu.sync_copy(hist, sp_hist.at[s])
    plsc.subcore_barrier()
    pltpu.sync_copy(sp_hist.at[s + 1 - 2 * half], phist)
    ustar, _ = _find_threshold(hist, phist, jnp.int32(TA))
    uthr = _I16((ustar + 1) << 19)

    def scan_chunk(buf_ref, cbase, off):
        def row_body(r, off):
            base = r * COLS
            ibase = cbase + base
            for sub in range(SUBV):
                p = buf_ref[pl.ds(base + sub * L, L)]
                m = _ukey(p) < uthr
                mi = m.astype(jnp.int32)
                pos = off + plsc.cumsum(mi) - 1
                m2 = jnp.logical_and(m, pos < _I16(CAP))
                plsc.store_scatter(cand_p, [pos], p, mask=m2)
                ipv = _I16(ibase + sub * L) + iota
                plsc.store_scatter(cand_i, [pos], ipv, mask=m2)
                off = off + plsc.all_reduce_population_count(m)
            return off

        return lax.fori_loop(0, QPC, row_body, off)

    off = jnp.zeros((L,), jnp.int32)
    for ci in range(NCHUNK):
        if ci > 0:
            (dma1 if ci % 2 == 1 else dma0).wait()
        off = scan_chunk(buf.at[ci % 2], wstart + ci * CHUNKP, off)
        if ci + 2 < NCHUNK:
            nxt = pltpu.async_copy(
                probs.at[pl.ds(pbase + wstart + (ci + 2) * CHUNKP, CHUNKP)],
                buf.at[ci % 2], sem0 if ci % 2 == 0 else sem1)
            if ci % 2 == 0:
                dma0 = nxt
            else:
                dma1 = nxt

    ncand = jnp.minimum(_scalar(off), jnp.int32(CAP))

    lax.fori_loop(0, HB // L, zero_hist, 0)

    def chist_body(j, _):
        p = cand_p[pl.ds(j * L, L)]
        m = (iota + j * L) < _I16(ncand)
        binv = lax.shift_right_logical(_ukey(p), _I16(19))
        plsc.addupdate_scatter(hist, [binv], ones_i, mask=m)
        return 0

    lax.fori_loop(0, (ncand + L - 1) // L, chist_body, 0)

    pltpu.sync_copy(hist, sp_hist.at[s])
    plsc.subcore_barrier()
    pltpu.sync_copy(sp_hist.at[s + 1 - 2 * half], phist)
    ustar2, _ = _find_threshold(hist, phist, jnp.int32(K))
    uthr2 = _I16((ustar2 + 1) << 19)

    def sent_fill(i, _):
        sv_p[pl.ds(i * L, L)] = _F16(-1.0)
        sv_i[pl.ds(i * L, L)] = _I16(0x7FFFFFF)
        return 0

    lax.fori_loop(0, 2 * SVCAP // L, sent_fill, 0)

    def sv_body(j, off):
        p = cand_p[pl.ds(j * L, L)]
        iv = cand_i[pl.ds(j * L, L)]
        m = jnp.logical_and(_ukey(p) < uthr2, (iota + j * L) < _I16(ncand))
        mi = m.astype(jnp.int32)
        pos = off + plsc.cumsum(mi) - 1
        m2 = jnp.logical_and(m, pos < _I16(SVCAP))
        plsc.store_scatter(sv_p, [pos], p, mask=m2)
        plsc.store_scatter(sv_i, [pos], iv, mask=m2)
        return off + plsc.all_reduce_population_count(m)

    svoff = lax.fori_loop(0, (ncand + L - 1) // L, sv_body,
                          jnp.zeros((L,), jnp.int32))
    svc = jnp.minimum(_scalar(svoff), jnp.int32(SVCAP))

    pltpu.sync_copy(sv_p.at[pl.ds(0, SVCAP)], sp_svp.at[s])
    pltpu.sync_copy(sv_i.at[pl.ds(0, SVCAP)], sp_svi.at[s])
    cnt_v[...] = jnp.full((L,), svc, jnp.int32)
    pltpu.sync_copy(cnt_v, sp_svc.at[s])
    plsc.subcore_barrier()

    @pl.when(half == 0)
    def _():
        svc8 = pl.multiple_of((svc + 7) & ~jnp.int32(7), 8)
        pltpu.sync_copy(sp_svp.at[s + 1], sv_p.at[pl.ds(svc8, SVCAP)])
        pltpu.sync_copy(sp_svi.at[s + 1], sv_i.at[pl.ds(svc8, SVCAP)])
        pltpu.sync_copy(sp_svc.at[s + 1], cnt_v)
        sloop = svc8 + jnp.int32(SVCAP)

        zz16 = jnp.zeros((L,), jnp.int32)
        for i in range(OUTP // L):
            ob_sc[pl.ds(i * L, L)] = _F16(0.0)
            ob_lb[pl.ds(i * L, L)] = _I16(0)
            ob_qd[pl.ds(i * L, L)] = _I16(0)
            rows = iota + i * L
            plsc.store_scatter(ob_sg, [rows * 2], _F16(0.0))
            plsc.store_scatter(ob_sg, [rows * 2 + 1], _F16(0.0))

        seg_dma.wait()
        vl = plsc.load_gather(ts_v, [_I16(0) + b])

        def rank_vreg(i, _):
            pv = sv_p[pl.ds(i * L, L)]
            iv = sv_i[pl.ds(i * L, L)]

            def jbody(j, cnt):
                js = _I16(0) + j
                pj = plsc.load_gather(sv_p, [js])
                ij = plsc.load_gather(sv_i, [js])
                beats = jnp.logical_or(
                    pj > pv, jnp.logical_and(pj == pv, ij < iv))
                return cnt + beats.astype(jnp.int32)

            r = lax.fori_loop(0, sloop, jbody, jnp.zeros((L,), jnp.int32))
            keep = jnp.logical_and(r < _I16(K), pv > _F16(-0.5))
            lab = jnp.bitwise_and(iv, _I16(COLS - 1))
            qid = lax.shift_right_logical(iv, _I16(8))
            plsc.store_scatter(ob_sc, [r], pv, mask=keep)
            plsc.store_scatter(ob_lb, [r], lab, mask=keep)
            plsc.store_scatter(ob_qd, [r], qid, mask=keep)
            sc_ = plsc.load_gather(seg_v, [qid * 2], mask=keep)
            sw_ = plsc.load_gather(seg_v, [qid * 2 + 1], mask=keep)
            t1 = (sc_ - 0.5 * sw_) * vl
            t2 = (sc_ + 0.5 * sw_) * vl
            plsc.store_scatter(ob_sg, [r * 2], t1, mask=keep)
            plsc.store_scatter(ob_sg, [r * 2 + 1], t2, mask=keep)
            return 0

        lax.fori_loop(0, (sloop + L - 1) // L, rank_vreg, 0)

        pltpu.sync_copy(ob_sc, o_sc.at[pl.ds(b * OUTP, OUTP)])
        pltpu.sync_copy(ob_lb, o_lb.at[pl.ds(b * OUTP, OUTP)])
        pltpu.sync_copy(ob_qd, o_qd.at[pl.ds(b * OUTP, OUTP)])
        pltpu.sync_copy(ob_sg, o_sg.at[pl.ds(b * 2 * OUTP, 2 * OUTP)])


@functools.partial(
    pl.kernel,
    out_type=[
        jax.ShapeDtypeStruct((B * OUTP,), jnp.float32),
        jax.ShapeDtypeStruct((B * OUTP,), jnp.int32),
        jax.ShapeDtypeStruct((B * OUTP * 2,), jnp.float32),
        jax.ShapeDtypeStruct((B * OUTP,), jnp.int32),
    ],
    mesh=plsc.VectorSubcoreMesh(core_axis_name="c", subcore_axis_name="s"),
    compiler_params=pltpu.CompilerParams(
        needs_layout_passes=False, use_tc_tiling_on_sc=False),
    scratch_types=[
        pltpu.VMEM((2, CHUNKP), jnp.float32),
        pltpu.VMEM((HB,), jnp.int32),
        pltpu.VMEM((HB,), jnp.int32),
        pltpu.VMEM((CAP,), jnp.float32),
        pltpu.VMEM((CAP,), jnp.int32),
        pltpu.VMEM((2 * SVCAP,), jnp.float32),
        pltpu.VMEM((2 * SVCAP,), jnp.int32),
        pltpu.VMEM((2 * Q,), jnp.float32),
        pltpu.VMEM((L,), jnp.float32),
        pltpu.VMEM((L,), jnp.int32),
        pltpu.VMEM((OUTP,), jnp.float32),
        pltpu.VMEM((OUTP,), jnp.int32),
        pltpu.VMEM((OUTP,), jnp.int32),
        pltpu.VMEM((OUTP * 2,), jnp.float32),
        pltpu.VMEM_SHARED((16, HB), jnp.int32),
        pltpu.VMEM_SHARED((16, SVCAP), jnp.float32),
        pltpu.VMEM_SHARED((16, SVCAP), jnp.int32),
        pltpu.VMEM_SHARED((16, L), jnp.int32),
        pltpu.SemaphoreType.DMA,
        pltpu.SemaphoreType.DMA,
        pltpu.SemaphoreType.DMA,
    ],
)
def _postprocess_sc(probs, segf, ts, o_sc, o_lb, o_sg, o_qd, *scratch):
    _sc_body(probs, segf, ts, o_sc, o_lb, o_sg, o_qd, *scratch)


def kernel(pred_logits, pred_segments, target_sizes):
    probs_pad = pl.pallas_call(
        _sigmoid_pad_body,
        out_shape=jax.ShapeDtypeStruct((B, Q, COLS), jnp.float32),
        grid=(B,),
        in_specs=[pl.BlockSpec((1, Q, C), lambda i: (i, 0, 0))],
        out_specs=pl.BlockSpec((1, Q, COLS), lambda i: (i, 0, 0)),
    )(pred_logits)
    probs = probs_pad.reshape(B * PADN)
    segf = pred_segments.reshape(B * 2 * Q)
    o_sc, o_lb, o_sg, o_qd = _postprocess_sc(probs, segf, target_sizes)
    o_sc = o_sc.reshape(B, OUTP)[:, :K]
    o_lb = o_lb.reshape(B, OUTP)[:, :K]
    o_sg = o_sg.reshape(B, OUTP, 2)[:, :K, :]
    o_qd = o_qd.reshape(B, OUTP)[:, :K]
    return (o_sc, o_lb, o_sg, o_qd)


# speedup vs baseline: 6.8375x; 1.3301x over previous
import functools

import jax
import jax.numpy as jnp
from jax import lax
from jax.experimental import pallas as pl
from jax.experimental.pallas import tpu as pltpu
from jax.experimental.pallas import tpu_sc as plsc

B, Q, C = 16, 5000, 200
K = 100
OUTP = 112
COLS = 256
SUBV = 13
PADN = Q * COLS
QP = 5120
WROWS = 2512
W1_ROW0 = Q - WROWS
OVERLAP = 2 * WROWS - Q
QPC = 157
CHUNKP = QPC * COLS
NCHUNK = WROWS // QPC
HB = 4096
CAP = 8192
SVCAP = 128
TA = 100
L = 16

_I16 = lambda v: jnp.full((L,), v, jnp.int32)
_F16 = lambda v: jnp.full((L,), v, jnp.float32)


def _scalar(x):
    return jnp.max(x)


def _ukey(p):
    return plsc.bitcast(_F16(1.0) - p, jnp.int32)


def _find_threshold(hist, phist, target):
    def body(i, carry):
        acc, ustar, found = carry
        hv = hist[pl.ds(i * L, L)] + phist[pl.ds(i * L, L)]
        cs = plsc.cumsum(hv)
        tot = jnp.max(cs)
        lane = jnp.sum((cs < (target - acc)).astype(jnp.int32))
        cross = jnp.logical_and(jnp.logical_not(found), acc + tot >= target)
        ustar = jnp.where(cross, i * L + lane, ustar)
        found = jnp.logical_or(found, cross)
        return acc + tot, ustar, found

    acc, ustar, found = lax.fori_loop(
        0, HB // L, body, (jnp.int32(0), jnp.int32(HB - 1), jnp.bool_(False)))
    return ustar, found


def _pre_body(x_ref, seg_ref, o_ref, rm_ref, cw_ref):
    sig = jax.nn.sigmoid(x_ref[0, :, :])
    o_ref[0, :, 0:C] = sig
    o_ref[0, :, C:COLS] = jnp.zeros((Q, COLS - C), jnp.float32)
    rm_ref[0, 0, 0:Q] = jnp.max(sig, axis=-1)
    rm_ref[0, 0, Q:QP] = jnp.zeros((QP - Q,), jnp.float32)
    cw_ref[0, 0, 0:Q] = seg_ref[0, :, 0]
    cw_ref[0, 0, Q:QP] = jnp.zeros((QP - Q,), jnp.float32)
    cw_ref[0, 1, 0:Q] = seg_ref[0, :, 1]
    cw_ref[0, 1, Q:QP] = jnp.zeros((QP - Q,), jnp.float32)


def _sc_body(probs, rm, cw, ts, o_sc, o_lb, o_sg, o_qd,
             buf, rm_v, hist, phist, cand_p, cand_i, sv_p, sv_i,
             c_v, w_v, ts_v, cnt_v,
             ob_sc, ob_lb, ob_qd, ob_sg,
             sp_hist, sp_svp, sp_svi, sp_svc,
             sem0, sem1, sem2):
    c = lax.axis_index("c")
    s = lax.axis_index("s")
    b = c * 8 + s // 2
    half = s % 2
    wrow0 = half * W1_ROW0
    wstart = wrow0 * COLS

    iota = lax.iota(jnp.int32, L)
    ones_i = jnp.ones((L,), jnp.int32)

    pbase = b * PADN
    dma0 = pltpu.async_copy(
        probs.at[pl.ds(pbase + wstart, CHUNKP)], buf.at[0], sem0)
    rm_dma = pltpu.async_copy(
        rm.at[pl.ds(b * QP + wrow0, WROWS)], rm_v, sem2)
    pltpu.sync_copy(ts, ts_v)
    dma0.wait()
    dma1 = pltpu.async_copy(
        probs.at[pl.ds(pbase + wstart + CHUNKP, CHUNKP)], buf.at[1], sem1)

    def zero_hist(i, _):
        hist[pl.ds(i * L, L)] = jnp.zeros((L,), jnp.int32)
        return 0

    lax.fori_loop(0, HB // L, zero_hist, 0)

    def hist_row(r, _):
        base = r * COLS
        for sub in range(SUBV):
            p = buf[0, pl.ds(base + sub * L, L)]
            binv = lax.shift_right_logical(_ukey(p), _I16(19))
            plsc.addupdate_scatter(hist, [binv], ones_i)
        return 0

    lax.fori_loop(0, QPC, hist_row, 0)

    pltpu.sync_copy(hist, sp_hist.at[s])
    plsc.subcore_barrier()
    pltpu.sync_copy(sp_hist.at[s + 1 - 2 * half], phist)
    ustar, _ = _find_threshold(hist, phist, jnp.int32(TA))
    uthr = _I16((ustar + 1) << 19)
    rm_dma.wait()

    def scan_chunk(buf_ref, chunk_row0, off):
        def row_body(rl, off):
            rloc = chunk_row0 + rl
            fl = plsc.load_gather(rm_v, [_I16(0) + rloc])
            hot = jnp.max((_ukey(fl) < uthr).astype(jnp.int32))
            valid = jnp.logical_or(half == 0, rloc >= OVERLAP)
            ok = jnp.logical_and(hot > 0, valid)

            def do(off):
                base = rl * COLS
                ibase = (wrow0 + rloc) * COLS
                for sub in range(SUBV):
                    p = buf_ref[pl.ds(base + sub * L, L)]
                    m = _ukey(p) < uthr
                    mi = m.astype(jnp.int32)
                    pos = off + plsc.cumsum(mi) - 1
                    m2 = jnp.logical_and(m, pos < _I16(CAP))
                    plsc.store_scatter(cand_p, [pos], p, mask=m2)
                    ipv = _I16(ibase + sub * L) + iota
                    plsc.store_scatter(cand_i, [pos], ipv, mask=m2)
                    off = off + plsc.all_reduce_population_count(m)
                return off

            return lax.cond(ok, do, lambda o: o, off)

        return lax.fori_loop(0, QPC, row_body, off)

    off = jnp.zeros((L,), jnp.int32)
    for ci in range(NCHUNK):
        if ci > 0:
            (dma1 if ci % 2 == 1 else dma0).wait()
        off = scan_chunk(buf.at[ci % 2], ci * QPC, off)
        if ci + 2 < NCHUNK:
            nxt = pltpu.async_copy(
                probs.at[pl.ds(pbase + wstart + (ci + 2) * CHUNKP, CHUNKP)],
                buf.at[ci % 2], sem0 if ci % 2 == 0 else sem1)
            if ci % 2 == 0:
                dma0 = nxt
            else:
                dma1 = nxt

    ncand = jnp.minimum(_scalar(off), jnp.int32(CAP))

    lax.fori_loop(0, HB // L, zero_hist, 0)

    def chist_body(j, _):
        p = cand_p[pl.ds(j * L, L)]
        m = (iota + j * L) < _I16(ncand)
        binv = lax.shift_right_logical(_ukey(p), _I16(19))
        plsc.addupdate_scatter(hist, [binv], ones_i, mask=m)
        return 0

    lax.fori_loop(0, (ncand + L - 1) // L, chist_body, 0)

    pltpu.sync_copy(hist, sp_hist.at[s])
    plsc.subcore_barrier()
    pltpu.sync_copy(sp_hist.at[s + 1 - 2 * half], phist)
    ustar2, _ = _find_threshold(hist, phist, jnp.int32(K))
    uthr2 = _I16((ustar2 + 1) << 19)

    def sent_fill(i, _):
        sv_p[pl.ds(i * L, L)] = _F16(-1.0)
        sv_i[pl.ds(i * L, L)] = _I16(0x7FFFFFF)
        return 0

    lax.fori_loop(0, 2 * SVCAP // L, sent_fill, 0)

    def sv_body(j, off):
        p = cand_p[pl.ds(j * L, L)]
        iv = cand_i[pl.ds(j * L, L)]
        m = jnp.logical_and(_ukey(p) < uthr2, (iota + j * L) < _I16(ncand))
        mi = m.astype(jnp.int32)
        pos = off + plsc.cumsum(mi) - 1
        m2 = jnp.logical_and(m, pos < _I16(SVCAP))
        plsc.store_scatter(sv_p, [pos], p, mask=m2)
        plsc.store_scatter(sv_i, [pos], iv, mask=m2)
        return off + plsc.all_reduce_population_count(m)

    svoff = lax.fori_loop(0, (ncand + L - 1) // L, sv_body,
                          jnp.zeros((L,), jnp.int32))
    svc = jnp.minimum(_scalar(svoff), jnp.int32(SVCAP))

    pltpu.sync_copy(sv_p.at[pl.ds(0, SVCAP)], sp_svp.at[s])
    pltpu.sync_copy(sv_i.at[pl.ds(0, SVCAP)], sp_svi.at[s])
    cnt_v[...] = jnp.full((L,), svc, jnp.int32)
    pltpu.sync_copy(cnt_v, sp_svc.at[s])
    plsc.subcore_barrier()

    @pl.when(half == 0)
    def _():
        svc8 = pl.multiple_of((svc + 7) & ~jnp.int32(7), 8)
        pltpu.sync_copy(sp_svp.at[s + 1], sv_p.at[pl.ds(svc8, SVCAP)])
        pltpu.sync_copy(sp_svi.at[s + 1], sv_i.at[pl.ds(svc8, SVCAP)])
        pltpu.sync_copy(sp_svc.at[s + 1], cnt_v)
        sloop = svc8 + jnp.int32(SVCAP)

        cw_dma0 = pltpu.async_copy(cw.at[pl.ds(b * 2 * QP, QP)], c_v, sem2)
        cw_dma1 = pltpu.async_copy(cw.at[pl.ds(b * 2 * QP + QP, QP)], w_v, sem0)
        for i in range(OUTP // L):
            ob_sc[pl.ds(i * L, L)] = _F16(0.0)
            ob_lb[pl.ds(i * L, L)] = _I16(0)
            ob_qd[pl.ds(i * L, L)] = _I16(0)
            rows = iota + i * L
            plsc.store_scatter(ob_sg, [rows * 2], _F16(0.0))
            plsc.store_scatter(ob_sg, [rows * 2 + 1], _F16(0.0))

        cw_dma0.wait()
        cw_dma1.wait()
        vl = plsc.load_gather(ts_v, [_I16(0) + b])

        def rank_vreg(i, _):
            pv = sv_p[pl.ds(i * L, L)]
            iv = sv_i[pl.ds(i * L, L)]

            def jbody(j, cnt):
                js = _I16(0) + j
                pj = plsc.load_gather(sv_p, [js])
                ij = plsc.load_gather(sv_i, [js])
                beats = jnp.logical_or(
                    pj > pv, jnp.logical_and(pj == pv, ij < iv))
                return cnt + beats.astype(jnp.int32)

            r = lax.fori_loop(0, sloop, jbody, jnp.zeros((L,), jnp.int32))
            keep = jnp.logical_and(r < _I16(K), pv > _F16(-0.5))
            lab = jnp.bitwise_and(iv, _I16(COLS - 1))
            qid = lax.shift_right_logical(iv, _I16(8))
            plsc.store_scatter(ob_sc, [r], pv, mask=keep)
            plsc.store_scatter(ob_lb, [r], lab, mask=keep)
            plsc.store_scatter(ob_qd, [r], qid, mask=keep)
            sc_ = plsc.load_gather(c_v, [qid], mask=keep)
            sw_ = plsc.load_gather(w_v, [qid], mask=keep)
            t1 = (sc_ - 0.5 * sw_) * vl
            t2 = (sc_ + 0.5 * sw_) * vl
            plsc.store_scatter(ob_sg, [r * 2], t1, mask=keep)
            plsc.store_scatter(ob_sg, [r * 2 + 1], t2, mask=keep)
            return 0

        lax.fori_loop(0, (sloop + L - 1) // L, rank_vreg, 0)

        pltpu.sync_copy(ob_sc, o_sc.at[pl.ds(b * OUTP, OUTP)])
        pltpu.sync_copy(ob_lb, o_lb.at[pl.ds(b * OUTP, OUTP)])
        pltpu.sync_copy(ob_qd, o_qd.at[pl.ds(b * OUTP, OUTP)])
        pltpu.sync_copy(ob_sg, o_sg.at[pl.ds(b * 2 * OUTP, 2 * OUTP)])


@functools.partial(
    pl.kernel,
    out_type=[
        jax.ShapeDtypeStruct((B * OUTP,), jnp.float32),
        jax.ShapeDtypeStruct((B * OUTP,), jnp.int32),
        jax.ShapeDtypeStruct((B * OUTP * 2,), jnp.float32),
        jax.ShapeDtypeStruct((B * OUTP,), jnp.int32),
    ],
    mesh=plsc.VectorSubcoreMesh(core_axis_name="c", subcore_axis_name="s"),
    compiler_params=pltpu.CompilerParams(
        needs_layout_passes=False, use_tc_tiling_on_sc=False),
    scratch_types=[
        pltpu.VMEM((2, CHUNKP), jnp.float32),
        pltpu.VMEM((WROWS,), jnp.float32),
        pltpu.VMEM((HB,), jnp.int32),
        pltpu.VMEM((HB,), jnp.int32),
        pltpu.VMEM((CAP,), jnp.float32),
        pltpu.VMEM((CAP,), jnp.int32),
        pltpu.VMEM((2 * SVCAP,), jnp.float32),
        pltpu.VMEM((2 * SVCAP,), jnp.int32),
        pltpu.VMEM((QP,), jnp.float32),
        pltpu.VMEM((QP,), jnp.float32),
        pltpu.VMEM((L,), jnp.float32),
        pltpu.VMEM((L,), jnp.int32),
        pltpu.VMEM((OUTP,), jnp.float32),
        pltpu.VMEM((OUTP,), jnp.int32),
        pltpu.VMEM((OUTP,), jnp.int32),
        pltpu.VMEM((OUTP * 2,), jnp.float32),
        pltpu.VMEM_SHARED((16, HB), jnp.int32),
        pltpu.VMEM_SHARED((16, SVCAP), jnp.float32),
        pltpu.VMEM_SHARED((16, SVCAP), jnp.int32),
        pltpu.VMEM_SHARED((16, L), jnp.int32),
        pltpu.SemaphoreType.DMA,
        pltpu.SemaphoreType.DMA,
        pltpu.SemaphoreType.DMA,
    ],
)
def _postprocess_sc(probs, rm, cw, ts, o_sc, o_lb, o_sg, o_qd, *scratch):
    _sc_body(probs, rm, cw, ts, o_sc, o_lb, o_sg, o_qd, *scratch)


def kernel(pred_logits, pred_segments, target_sizes):
    probs_pad, rm_pad, cw_pad = pl.pallas_call(
        _pre_body,
        out_shape=[
            jax.ShapeDtypeStruct((B, Q, COLS), jnp.float32),
            jax.ShapeDtypeStruct((B, 1, QP), jnp.float32),
            jax.ShapeDtypeStruct((B, 2, QP), jnp.float32),
        ],
        grid=(B,),
        in_specs=[
            pl.BlockSpec((1, Q, C), lambda i: (i, 0, 0)),
            pl.BlockSpec((1, Q, 2), lambda i: (i, 0, 0)),
        ],
        out_specs=[
            pl.BlockSpec((1, Q, COLS), lambda i: (i, 0, 0)),
            pl.BlockSpec((1, 1, QP), lambda i: (i, 0, 0)),
            pl.BlockSpec((1, 2, QP), lambda i: (i, 0, 0)),
        ],
    )(pred_logits, pred_segments)
    probs = probs_pad.reshape(B * PADN)
    rm = rm_pad.reshape(B * QP)
    cwf = cw_pad.reshape(B * 2 * QP)
    o_sc, o_lb, o_sg, o_qd = _postprocess_sc(probs, rm, cwf, target_sizes)
    o_sc = o_sc.reshape(B, OUTP)[:, :K]
    o_lb = o_lb.reshape(B, OUTP)[:, :K]
    o_sg = o_sg.reshape(B, OUTP, 2)[:, :K, :]
    o_qd = o_qd.reshape(B, OUTP)[:, :K]
    return (o_sc, o_lb, o_sg, o_qd)


# speedup vs baseline: 6.9071x; 1.0102x over previous
import functools

import jax
import jax.numpy as jnp
from jax import lax
from jax.experimental import pallas as pl
from jax.experimental.pallas import tpu as pltpu
from jax.experimental.pallas import tpu_sc as plsc

B, Q, C = 16, 5000, 200
K = 100
OUTP = 112
COLS = 256
SUBV = 13
PADN = Q * COLS
QP = 5120
WROWS = 2512
W1_ROW0 = Q - WROWS
OVERLAP = 2 * WROWS - Q
QPC = 157
CHUNKP = QPC * COLS
NCHUNK = WROWS // QPC
HB = 4096
CAP = 8192
SVCAP = 128
TA = 100
L = 16

_I16 = lambda v: jnp.full((L,), v, jnp.int32)
_F16 = lambda v: jnp.full((L,), v, jnp.float32)


def _scalar(x):
    return jnp.max(x)


def _ukey(p):
    return plsc.bitcast(_F16(1.0) - p, jnp.int32)


def _find_threshold(hist, phist, target):
    def body(i, carry):
        acc, ustar, found = carry
        hv = hist[pl.ds(i * L, L)] + phist[pl.ds(i * L, L)]
        cs = plsc.cumsum(hv)
        tot = jnp.max(cs)
        lane = jnp.sum((cs < (target - acc)).astype(jnp.int32))
        cross = jnp.logical_and(jnp.logical_not(found), acc + tot >= target)
        ustar = jnp.where(cross, i * L + lane, ustar)
        found = jnp.logical_or(found, cross)
        return acc + tot, ustar, found

    acc, ustar, found = lax.fori_loop(
        0, HB // L, body, (jnp.int32(0), jnp.int32(HB - 1), jnp.bool_(False)))
    return ustar, found


def _pre_body(x_ref, seg_ref, o_ref, rm_ref, cw_ref):
    sig = jax.nn.sigmoid(x_ref[0, :, :])
    o_ref[0, :, 0:C] = sig
    o_ref[0, :, C:COLS] = jnp.zeros((Q, COLS - C), jnp.float32)
    zpad = jnp.zeros((QP - Q,), jnp.float32)
    rm_ref[0] = jnp.concatenate(
        [jnp.max(sig, axis=-1), zpad]).reshape(QP // 128, 128)
    cw_ref[0, 0] = jnp.concatenate(
        [seg_ref[0, :, 0], zpad]).reshape(QP // 128, 128)
    cw_ref[0, 1] = jnp.concatenate(
        [seg_ref[0, :, 1], zpad]).reshape(QP // 128, 128)


def _sc_body(probs, rm, cw, ts, o_sc, o_lb, o_sg, o_qd,
             buf, rm_v, hist, phist, cand_p, cand_i, sv_p, sv_i,
             c_v, w_v, ts_v, cnt_v,
             ob_sc, ob_lb, ob_qd, ob_sg,
             sp_hist, sp_svp, sp_svi, sp_svc,
             sem0, sem1, sem2):
    c = lax.axis_index("c")
    s = lax.axis_index("s")
    b = c * 8 + s // 2
    half = s % 2
    wrow0 = half * W1_ROW0
    wstart = wrow0 * COLS

    iota = lax.iota(jnp.int32, L)
    ones_i = jnp.ones((L,), jnp.int32)

    pbase = b * PADN
    dma0 = pltpu.async_copy(
        probs.at[pl.ds(pbase + wstart, CHUNKP)], buf.at[0], sem0)
    rm_dma = pltpu.async_copy(
        rm.at[pl.ds(b * QP + wrow0, WROWS)], rm_v, sem2)
    pltpu.sync_copy(ts, ts_v)
    dma0.wait()
    dma1 = pltpu.async_copy(
        probs.at[pl.ds(pbase + wstart + CHUNKP, CHUNKP)], buf.at[1], sem1)

    def zero_hist(i, _):
        hist[pl.ds(i * L, L)] = jnp.zeros((L,), jnp.int32)
        return 0

    lax.fori_loop(0, HB // L, zero_hist, 0)

    def hist_row(r, _):
        base = r * COLS
        for sub in range(SUBV):
            p = buf[0, pl.ds(base + sub * L, L)]
            binv = lax.shift_right_logical(_ukey(p), _I16(19))
            plsc.addupdate_scatter(hist, [binv], ones_i)
        return 0

    lax.fori_loop(0, QPC, hist_row, 0)

    pltpu.sync_copy(hist, sp_hist.at[s])
    plsc.subcore_barrier()
    pltpu.sync_copy(sp_hist.at[s + 1 - 2 * half], phist)
    ustar, _ = _find_threshold(hist, phist, jnp.int32(TA))
    uthr = _I16((ustar + 1) << 19)
    rm_dma.wait()

    def scan_chunk(buf_ref, chunk_row0, off):
        def row_body(rl, off):
            rloc = chunk_row0 + rl
            fl = plsc.load_gather(rm_v, [_I16(0) + rloc])
            hot = jnp.max((_ukey(fl) < uthr).astype(jnp.int32))
            valid = jnp.logical_or(half == 0, rloc >= OVERLAP)
            ok = jnp.logical_and(hot > 0, valid)

            def do(off):
                base = rl * COLS
                ibase = (wrow0 + rloc) * COLS
                for sub in range(SUBV):
                    p = buf_ref[pl.ds(base + sub * L, L)]
                    m = _ukey(p) < uthr
                    mi = m.astype(jnp.int32)
                    pos = off + plsc.cumsum(mi) - 1
                    m2 = jnp.logical_and(m, pos < _I16(CAP))
                    plsc.store_scatter(cand_p, [pos], p, mask=m2)
                    ipv = _I16(ibase + sub * L) + iota
                    plsc.store_scatter(cand_i, [pos], ipv, mask=m2)
                    off = off + plsc.all_reduce_population_count(m)
                return off

            return lax.cond(ok, do, lambda o: o, off)

        return lax.fori_loop(0, QPC, row_body, off)

    off = jnp.zeros((L,), jnp.int32)
    for ci in range(NCHUNK):
        if ci > 0:
            (dma1 if ci % 2 == 1 else dma0).wait()
        off = scan_chunk(buf.at[ci % 2], ci * QPC, off)
        if ci + 2 < NCHUNK:
            nxt = pltpu.async_copy(
                probs.at[pl.ds(pbase + wstart + (ci + 2) * CHUNKP, CHUNKP)],
                buf.at[ci % 2], sem0 if ci % 2 == 0 else sem1)
            if ci % 2 == 0:
                dma0 = nxt
            else:
                dma1 = nxt

    ncand = jnp.minimum(_scalar(off), jnp.int32(CAP))

    lax.fori_loop(0, HB // L, zero_hist, 0)

    def chist_body(j, _):
        p = cand_p[pl.ds(j * L, L)]
        m = (iota + j * L) < _I16(ncand)
        binv = lax.shift_right_logical(_ukey(p), _I16(19))
        plsc.addupdate_scatter(hist, [binv], ones_i, mask=m)
        return 0

    lax.fori_loop(0, (ncand + L - 1) // L, chist_body, 0)

    pltpu.sync_copy(hist, sp_hist.at[s])
    plsc.subcore_barrier()
    pltpu.sync_copy(sp_hist.at[s + 1 - 2 * half], phist)
    ustar2, _ = _find_threshold(hist, phist, jnp.int32(K))
    uthr2 = _I16((ustar2 + 1) << 19)

    def sent_fill(i, _):
        sv_p[pl.ds(i * L, L)] = _F16(-1.0)
        sv_i[pl.ds(i * L, L)] = _I16(0x7FFFFFF)
        return 0

    lax.fori_loop(0, 2 * SVCAP // L, sent_fill, 0)

    def sv_body(j, off):
        p = cand_p[pl.ds(j * L, L)]
        iv = cand_i[pl.ds(j * L, L)]
        m = jnp.logical_and(_ukey(p) < uthr2, (iota + j * L) < _I16(ncand))
        mi = m.astype(jnp.int32)
        pos = off + plsc.cumsum(mi) - 1
        m2 = jnp.logical_and(m, pos < _I16(SVCAP))
        plsc.store_scatter(sv_p, [pos], p, mask=m2)
        plsc.store_scatter(sv_i, [pos], iv, mask=m2)
        return off + plsc.all_reduce_population_count(m)

    svoff = lax.fori_loop(0, (ncand + L - 1) // L, sv_body,
                          jnp.zeros((L,), jnp.int32))
    svc = jnp.minimum(_scalar(svoff), jnp.int32(SVCAP))

    pltpu.sync_copy(sv_p.at[pl.ds(0, SVCAP)], sp_svp.at[s])
    pltpu.sync_copy(sv_i.at[pl.ds(0, SVCAP)], sp_svi.at[s])
    cnt_v[...] = jnp.full((L,), svc, jnp.int32)
    pltpu.sync_copy(cnt_v, sp_svc.at[s])
    plsc.subcore_barrier()

    @pl.when(half == 0)
    def _():
        svc8 = pl.multiple_of((svc + 7) & ~jnp.int32(7), 8)
        pltpu.sync_copy(sp_svp.at[s + 1], sv_p.at[pl.ds(svc8, SVCAP)])
        pltpu.sync_copy(sp_svi.at[s + 1], sv_i.at[pl.ds(svc8, SVCAP)])
        pltpu.sync_copy(sp_svc.at[s + 1], cnt_v)
        pcnt = jnp.minimum(jnp.max(cnt_v[...]), jnp.int32(SVCAP))
        sloop = svc8 + ((pcnt + 7) & ~jnp.int32(7))

        cw_dma0 = pltpu.async_copy(cw.at[pl.ds(b * 2 * QP, QP)], c_v, sem2)
        cw_dma1 = pltpu.async_copy(cw.at[pl.ds(b * 2 * QP + QP, QP)], w_v, sem0)
        for i in range(OUTP // L):
            ob_sc[pl.ds(i * L, L)] = _F16(0.0)
            ob_lb[pl.ds(i * L, L)] = _I16(0)
            ob_qd[pl.ds(i * L, L)] = _I16(0)
            rows = iota + i * L
            plsc.store_scatter(ob_sg, [rows * 2], _F16(0.0))
            plsc.store_scatter(ob_sg, [rows * 2 + 1], _F16(0.0))

        cw_dma0.wait()
        cw_dma1.wait()
        vl = plsc.load_gather(ts_v, [_I16(0) + b])

        def rank_vreg(i, _):
            pv = sv_p[pl.ds(i * L, L)]
            iv = sv_i[pl.ds(i * L, L)]

            def jbody(j, cnt):
                js = _I16(0) + j
                pj = plsc.load_gather(sv_p, [js])
                ij = plsc.load_gather(sv_i, [js])
                beats = jnp.logical_or(
                    pj > pv, jnp.logical_and(pj == pv, ij < iv))
                return cnt + beats.astype(jnp.int32)

            r = lax.fori_loop(0, sloop, jbody, jnp.zeros((L,), jnp.int32))
            keep = jnp.logical_and(r < _I16(K), pv > _F16(-0.5))
            lab = jnp.bitwise_and(iv, _I16(COLS - 1))
            qid = lax.shift_right_logical(iv, _I16(8))
            plsc.store_scatter(ob_sc, [r], pv, mask=keep)
            plsc.store_scatter(ob_lb, [r], lab, mask=keep)
            plsc.store_scatter(ob_qd, [r], qid, mask=keep)
            sc_ = plsc.load_gather(c_v, [qid], mask=keep)
            sw_ = plsc.load_gather(w_v, [qid], mask=keep)
            t1 = (sc_ - 0.5 * sw_) * vl
            t2 = (sc_ + 0.5 * sw_) * vl
            plsc.store_scatter(ob_sg, [r * 2], t1, mask=keep)
            plsc.store_scatter(ob_sg, [r * 2 + 1], t2, mask=keep)
            return 0

        lax.fori_loop(0, (sloop + L - 1) // L, rank_vreg, 0)

        pltpu.sync_copy(ob_sc, o_sc.at[pl.ds(b * OUTP, OUTP)])
        pltpu.sync_copy(ob_lb, o_lb.at[pl.ds(b * OUTP, OUTP)])
        pltpu.sync_copy(ob_qd, o_qd.at[pl.ds(b * OUTP, OUTP)])
        pltpu.sync_copy(ob_sg, o_sg.at[pl.ds(b * 2 * OUTP, 2 * OUTP)])


@functools.partial(
    pl.kernel,
    out_type=[
        jax.ShapeDtypeStruct((B * OUTP,), jnp.float32),
        jax.ShapeDtypeStruct((B * OUTP,), jnp.int32),
        jax.ShapeDtypeStruct((B * OUTP * 2,), jnp.float32),
        jax.ShapeDtypeStruct((B * OUTP,), jnp.int32),
    ],
    mesh=plsc.VectorSubcoreMesh(core_axis_name="c", subcore_axis_name="s"),
    compiler_params=pltpu.CompilerParams(
        needs_layout_passes=False, use_tc_tiling_on_sc=False),
    scratch_types=[
        pltpu.VMEM((2, CHUNKP), jnp.float32),
        pltpu.VMEM((WROWS,), jnp.float32),
        pltpu.VMEM((HB,), jnp.int32),
        pltpu.VMEM((HB,), jnp.int32),
        pltpu.VMEM((CAP,), jnp.float32),
        pltpu.VMEM((CAP,), jnp.int32),
        pltpu.VMEM((2 * SVCAP,), jnp.float32),
        pltpu.VMEM((2 * SVCAP,), jnp.int32),
        pltpu.VMEM((QP,), jnp.float32),
        pltpu.VMEM((QP,), jnp.float32),
        pltpu.VMEM((L,), jnp.float32),
        pltpu.VMEM((L,), jnp.int32),
        pltpu.VMEM((OUTP,), jnp.float32),
        pltpu.VMEM((OUTP,), jnp.int32),
        pltpu.VMEM((OUTP,), jnp.int32),
        pltpu.VMEM((OUTP * 2,), jnp.float32),
        pltpu.VMEM_SHARED((16, HB), jnp.int32),
        pltpu.VMEM_SHARED((16, SVCAP), jnp.float32),
        pltpu.VMEM_SHARED((16, SVCAP), jnp.int32),
        pltpu.VMEM_SHARED((16, L), jnp.int32),
        pltpu.SemaphoreType.DMA,
        pltpu.SemaphoreType.DMA,
        pltpu.SemaphoreType.DMA,
    ],
)
def _postprocess_sc(probs, rm, cw, ts, o_sc, o_lb, o_sg, o_qd, *scratch):
    _sc_body(probs, rm, cw, ts, o_sc, o_lb, o_sg, o_qd, *scratch)


def kernel(pred_logits, pred_segments, target_sizes):
    probs_pad, rm_pad, cw_pad = pl.pallas_call(
        _pre_body,
        out_shape=[
            jax.ShapeDtypeStruct((B, Q, COLS), jnp.float32),
            jax.ShapeDtypeStruct((B, QP // 128, 128), jnp.float32),
            jax.ShapeDtypeStruct((B, 2, QP // 128, 128), jnp.float32),
        ],
        grid=(B,),
        in_specs=[
            pl.BlockSpec((1, Q, C), lambda i: (i, 0, 0)),
            pl.BlockSpec((1, Q, 2), lambda i: (i, 0, 0)),
        ],
        out_specs=[
            pl.BlockSpec((1, Q, COLS), lambda i: (i, 0, 0)),
            pl.BlockSpec((1, QP // 128, 128), lambda i: (i, 0, 0)),
            pl.BlockSpec((1, 2, QP // 128, 128), lambda i: (i, 0, 0, 0)),
        ],
    )(pred_logits, pred_segments)
    probs = probs_pad.reshape(B * PADN)
    rm = rm_pad.reshape(B * QP)
    cwf = cw_pad.reshape(B * 2 * QP)
    o_sc, o_lb, o_sg, o_qd = _postprocess_sc(probs, rm, cwf, target_sizes)
    o_sc = o_sc.reshape(B, OUTP)[:, :K]
    o_lb = o_lb.reshape(B, OUTP)[:, :K]
    o_sg = o_sg.reshape(B, OUTP, 2)[:, :K, :]
    o_qd = o_qd.reshape(B, OUTP)[:, :K]
    return (o_sc, o_lb, o_sg, o_qd)
